# Initial kernel scaffold; baseline (speedup 1.0000x reference)
import jax
import jax.numpy as jnp
from jax.experimental import pallas as pl


def kernel(tokens, table):
    raise NotImplementedError("write your pallas kernel here")


# speedup vs baseline: 6.8138x; 6.8138x over previous
import math

import jax
import jax.numpy as jnp
from jax.experimental import pallas as pl
from jax.experimental.pallas import tpu as pltpu
from jax.experimental.pallas import tpu_sc as plsc

_EMB = 128
_SCALE = math.sqrt(_EMB)
_W = 128


def _scale_body(t_ref, o_ref):
    o_ref[...] = t_ref[...] * _SCALE


def _scale_table(table):
    rows = table.shape[0]
    blk = 5000
    return pl.pallas_call(
        _scale_body,
        grid=(rows // blk,),
        in_specs=[pl.BlockSpec((blk, _EMB), lambda i: (i, 0))],
        out_specs=pl.BlockSpec((blk, _EMB), lambda i: (i, 0)),
        out_shape=jax.ShapeDtypeStruct(table.shape, table.dtype),
    )(table)


def _sc_gather(table, flat_idx):
    n = flat_idx.shape[0]
    idx2 = flat_idx.reshape(1, n)
    mesh = plsc.VectorSubcoreMesh(core_axis_name="c", subcore_axis_name="s")

    @pl.kernel(
        out_type=jax.ShapeDtypeStruct((n, _EMB), table.dtype),
        mesh=mesh,
    )
    def k(tab_hbm, i_hbm, o_hbm):
        def body(i_vmem, o_vmem):
            pltpu.sync_copy(tab_hbm.at[i_vmem.at[0]], o_vmem)

        pltpu.emit_pipeline(
            body,
            grid=(n // _W,),
            in_specs=[pl.BlockSpec((1, _W), lambda i: (0, i))],
            out_specs=[pl.BlockSpec((_W, _EMB), lambda i: (i, 0))],
            core_axis_name=("c", "s"),
            dimension_semantics=(pltpu.PARALLEL,),
        )(i_hbm, o_hbm)

    return k(table, idx2)


def kernel(tokens, table):
    scaled = _scale_table(table)
    flat = tokens.reshape(-1)
    out = _sc_gather(scaled, flat)
    return out.reshape(*tokens.shape, _EMB)


# speedup vs baseline: 8.2515x; 1.2110x over previous
import math

import jax
import jax.numpy as jnp
from jax.experimental import pallas as pl
from jax.experimental.pallas import tpu as pltpu
from jax.experimental.pallas import tpu_sc as plsc

_EMB = 128
_SCALE = math.sqrt(_EMB)
_W = 256
_SUB = 128


def _scale_body(t_ref, o_ref):
    o_ref[...] = t_ref[...] * _SCALE


def _scale_table(table):
    rows = table.shape[0]
    blk = 5000
    return pl.pallas_call(
        _scale_body,
        grid=(rows // blk,),
        in_specs=[pl.BlockSpec((blk, _EMB), lambda i: (i, 0))],
        out_specs=pl.BlockSpec((blk, _EMB), lambda i: (i, 0)),
        out_shape=jax.ShapeDtypeStruct(table.shape, table.dtype),
    )(table)


def _sc_gather(table, flat_idx):
    n = flat_idx.shape[0]
    idx2 = flat_idx.reshape(1, n)
    mesh = plsc.VectorSubcoreMesh(core_axis_name="c", subcore_axis_name="s")

    @pl.kernel(
        out_type=jax.ShapeDtypeStruct((n, _EMB), table.dtype),
        mesh=mesh,
        scratch_types=[pltpu.SemaphoreType.DMA],
    )
    def k(tab_hbm, i_hbm, o_hbm, sem):
        def body(i_vmem, o_vmem):
            cps = [
                pltpu.async_copy(
                    tab_hbm.at[i_vmem.at[0, pl.ds(j * _SUB, _SUB)]],
                    o_vmem.at[pl.ds(j * _SUB, _SUB), :],
                    sem,
                )
                for j in range(_W // _SUB)
            ]
            for cp in cps:
                cp.wait()

        pltpu.emit_pipeline(
            body,
            grid=(n // _W,),
            in_specs=[pl.BlockSpec((1, _W), lambda i: (0, i))],
            out_specs=[pl.BlockSpec((_W, _EMB), lambda i: (i, 0))],
            core_axis_name=("c", "s"),
            dimension_semantics=(pltpu.PARALLEL,),
        )(i_hbm, o_hbm)

    return k(table, idx2)


def kernel(tokens, table):
    scaled = _scale_table(table)
    flat = tokens.reshape(-1)
    out = _sc_gather(scaled, flat)
    return out.reshape(*tokens.shape, _EMB)


# speedup vs baseline: 8.2556x; 1.0005x over previous
import math

import jax
import jax.numpy as jnp
from jax import lax
from jax.experimental import pallas as pl
from jax.experimental.pallas import tpu as pltpu
from jax.experimental.pallas import tpu_sc as plsc

_EMB = 128
_SCALE = math.sqrt(_EMB)
_K = 128
_NBUF = 4


def _scale_body(t_ref, o_ref):
    o_ref[...] = t_ref[...] * _SCALE


def _scale_table(table):
    rows = table.shape[0]
    blk = 5000
    return pl.pallas_call(
        _scale_body,
        grid=(rows // blk,),
        in_specs=[pl.BlockSpec((blk, _EMB), lambda i: (i, 0))],
        out_specs=pl.BlockSpec((blk, _EMB), lambda i: (i, 0)),
        out_shape=jax.ShapeDtypeStruct(table.shape, table.dtype),
    )(table)


def _sc_gather(table, flat_idx):
    n = flat_idx.shape[0]
    mesh = plsc.VectorSubcoreMesh(core_axis_name="c", subcore_axis_name="s")
    nw = 32
    per_w = n // nw
    nch = per_w // _K
    nrounds = nch // _NBUF
    assert nch % _NBUF == 0

    @pl.kernel(
        out_type=jax.ShapeDtypeStruct((n, _EMB), table.dtype),
        mesh=mesh,
        scratch_types=[
            pltpu.VMEM((per_w,), jnp.int32),
            *[pltpu.VMEM((_K, _EMB), jnp.float32) for _ in range(_NBUF)],
            pltpu.SemaphoreType.DMA,
            *[pltpu.SemaphoreType.DMA for _ in range(_NBUF)],
            *[pltpu.SemaphoreType.DMA for _ in range(_NBUF)],
        ],
    )
    def k(tab_hbm, i_hbm, o_hbm, idx_v, *rest):
        bufs = rest[:_NBUF]
        isem = rest[_NBUF]
        gsems = rest[_NBUF + 1 : 2 * _NBUF + 1]
        wsems = rest[2 * _NBUF + 1 :]
        wid = lax.axis_index("s") * 2 + lax.axis_index("c")
        base = wid * per_w
        pltpu.async_copy(i_hbm.at[pl.ds(base, per_w)], idx_v, isem).wait()

        def issue_gather(g, b):
            pltpu.async_copy(
                tab_hbm.at[idx_v.at[pl.ds(g * _K, _K)]], bufs[b], gsems[b]
            )

        def issue_write(g, b):
            pltpu.async_copy(
                bufs[b], o_hbm.at[pl.ds(base + g * _K, _K)], wsems[b]
            )

        def wait_gather(b):
            pltpu.make_async_copy(tab_hbm.at[pl.ds(0, _K)], bufs[b], gsems[b]).wait()

        def wait_write(b):
            pltpu.make_async_copy(bufs[b], o_hbm.at[pl.ds(base, _K)], wsems[b]).wait()

        @pl.loop(0, nrounds)
        def _(r):
            for b in range(_NBUF):
                @pl.when(r > 0)
                def _():
                    wait_write(b)
                issue_gather(r * _NBUF + b, b)
            for b in range(_NBUF):
                wait_gather(b)
                issue_write(r * _NBUF + b, b)

        for b in range(_NBUF):
            wait_write(b)

    return k(table, flat_idx)


def kernel(tokens, table):
    scaled = _scale_table(table)
    flat = tokens.reshape(-1)
    out = _sc_gather(scaled, flat)
    return out.reshape(*tokens.shape, _EMB)


# speedup vs baseline: 8.2891x; 1.0041x over previous
import math

import jax
import jax.numpy as jnp
from jax import lax
from jax.experimental import pallas as pl
from jax.experimental.pallas import tpu as pltpu
from jax.experimental.pallas import tpu_sc as plsc

_EMB = 128
_SCALE = math.sqrt(_EMB)
_K = 128
_NBUF = 5


def _scale_body(t_ref, o_ref):
    o_ref[...] = t_ref[...] * _SCALE


def _scale_table(table):
    rows = table.shape[0]
    blk = 10000
    return pl.pallas_call(
        _scale_body,
        grid=(rows // blk,),
        in_specs=[pl.BlockSpec((blk, _EMB), lambda i: (i, 0))],
        out_specs=pl.BlockSpec((blk, _EMB), lambda i: (i, 0)),
        out_shape=jax.ShapeDtypeStruct(table.shape, table.dtype),
    )(table)


def _sc_gather(table, flat_idx):
    n = flat_idx.shape[0]
    mesh = plsc.VectorSubcoreMesh(core_axis_name="c", subcore_axis_name="s")
    nw = 32
    per_w = n // nw
    nch = per_w // _K
    nrounds = nch // _NBUF
    assert nch % _NBUF == 0

    @pl.kernel(
        out_type=jax.ShapeDtypeStruct((n, _EMB), table.dtype),
        mesh=mesh,
        scratch_types=[
            pltpu.VMEM((per_w,), jnp.int32),
            *[pltpu.VMEM((_K, _EMB), jnp.float32) for _ in range(_NBUF)],
            pltpu.SemaphoreType.DMA,
            *[pltpu.SemaphoreType.DMA for _ in range(_NBUF)],
            *[pltpu.SemaphoreType.DMA for _ in range(_NBUF)],
        ],
    )
    def k(tab_hbm, i_hbm, o_hbm, idx_v, *rest):
        bufs = rest[:_NBUF]
        isem = rest[_NBUF]
        gsems = rest[_NBUF + 1 : 2 * _NBUF + 1]
        wsems = rest[2 * _NBUF + 1 :]
        wid = lax.axis_index("s") * 2 + lax.axis_index("c")
        base = wid * per_w
        pltpu.async_copy(i_hbm.at[pl.ds(base, per_w)], idx_v, isem).wait()

        def issue_gather(g, b):
            pltpu.async_copy(
                tab_hbm.at[idx_v.at[pl.ds(g * _K, _K)]], bufs[b], gsems[b]
            )

        def issue_write(g, b):
            pltpu.async_copy(
                bufs[b], o_hbm.at[pl.ds(base + g * _K, _K)], wsems[b]
            )

        def wait_gather(b):
            pltpu.make_async_copy(tab_hbm.at[pl.ds(0, _K)], bufs[b], gsems[b]).wait()

        def wait_write(b):
            pltpu.make_async_copy(bufs[b], o_hbm.at[pl.ds(base, _K)], wsems[b]).wait()

        @pl.loop(0, nrounds)
        def _(r):
            for b in range(_NBUF):
                @pl.when(r > 0)
                def _():
                    wait_write(b)
                issue_gather(r * _NBUF + b, b)
            for b in range(_NBUF):
                wait_gather(b)
                issue_write(r * _NBUF + b, b)

        for b in range(_NBUF):
            wait_write(b)

    return k(table, flat_idx)


def kernel(tokens, table):
    scaled = _scale_table(table)
    flat = tokens.reshape(-1)
    out = _sc_gather(scaled, flat)
    return out.reshape(*tokens.shape, _EMB)
